# Initial kernel scaffold; baseline (speedup 1.0000x reference)
import functools

import jax
import jax.numpy as jnp
from jax import lax
from jax.experimental import pallas as pl
from jax.experimental.pallas import tpu as pltpu
from jax.experimental.pallas import tpu_sc as plsc

N_NODES = 10000
N_EDGES = 320000
HIDDEN = 128

NC = 2
NS = 16
L = 16

EPT1 = N_EDGES // NS
EPT2 = N_EDGES // (NC * NS)
CHUNK = 128
ROWS = -(-EPT1 // CHUNK)
EPT1_PAD = ROWS * CHUNK
S_PAD = 10240
ZSLICE = S_PAD // NS
PARAMS = 528

_mesh = plsc.VectorSubcoreMesh(core_axis_name="c", subcore_axis_name="s")


@functools.partial(
    pl.kernel,
    out_type=jax.ShapeDtypeStruct((N_EDGES,), jnp.float32),
    mesh=_mesh,
    scratch_types=[
        pltpu.VMEM((N_NODES,), jnp.float32),
        pltpu.VMEM((EPT1_PAD,), jnp.int32),
        pltpu.VMEM((EPT1_PAD,), jnp.int32),
        pltpu.VMEM((ROWS, CHUNK), jnp.int32),
        pltpu.VMEM((EPT1_PAD,), jnp.float32),
        pltpu.VMEM((N_NODES,), jnp.float32),
        pltpu.VMEM((EPT2,), jnp.float32),
        pltpu.VMEM((S_PAD // NS,), jnp.float32),
        pltpu.VMEM((PARAMS,), jnp.float32),
        pltpu.VMEM_SHARED((S_PAD,), jnp.float32),
        pltpu.SemaphoreType.DMA,
    ],
)
def _sc_kernel(x_hbm, srcp_hbm, dstp_hbm, dst3_hbm, par_hbm, out_hbm,
               x_v, src_v, dst_v, dsti_v, vals_v, s_v, out_v, zb_v, par_v,
               s_sh, sem):
    c_id = lax.axis_index("c")
    s_id = lax.axis_index("s")
    wid = 2 * s_id + c_id

    pltpu.sync_copy(x_hbm, x_v)
    pltpu.sync_copy(srcp_hbm.at[s_id], src_v)
    pltpu.sync_copy(dstp_hbm.at[s_id], dst_v)
    pltpu.sync_copy(dst3_hbm.at[s_id], dsti_v)
    pltpu.sync_copy(par_hbm, par_v)

    zeros = jnp.zeros((L,), jnp.float32)
    for k in range(ZSLICE // L):
        zb_v[pl.ds(k * L, L)] = zeros
    pltpu.sync_copy(zb_v, s_sh.at[pl.ds(s_id * ZSLICE, ZSLICE)])

    acc_a = jnp.zeros((L,), jnp.float32)
    acc_b = jnp.zeros((L,), jnp.float32)
    acc_c = jnp.zeros((L,), jnp.float32)
    for j in range(HIDDEN // L):
        w = par_v[pl.ds(j * L, L)]
        w1 = par_v[pl.ds(HIDDEN + j * L, L)]
        w2 = par_v[pl.ds(2 * HIDDEN + j * L, L)]
        bi = par_v[pl.ds(3 * HIDDEN + j * L, L)]
        acc_a = acc_a + w * w1
        acc_b = acc_b + w * w2
        acc_c = acc_c + bi * (w1 + w2)
    acc_c = acc_c + par_v[pl.ds(4 * HIDDEN, L)]
    a = jnp.sum(acc_a)
    b = jnp.sum(acc_b)
    c = jnp.sum(acc_c)

    def gather_body(i, _):
        idx = src_v[pl.ds(i * L, L)]
        vals_v[pl.ds(i * L, L)] = plsc.load_gather(x_v, [idx])
        return 0

    lax.fori_loop(0, EPT1_PAD // L, gather_body, 0)

    plsc.subcore_barrier()

    DEPTH = 8
    NFULL = ROWS // DEPTH

    def scatter_chunk(jc, _):
        descs = []
        for k in range(DEPTH):
            j = jc * DEPTH + k
            descs.append(
                pltpu.async_copy(
                    vals_v.at[pl.ds(j * CHUNK, CHUNK)],
                    s_sh.at[dsti_v.at[j]],
                    sem,
                    add=True,
                )
            )
        for d in descs:
            d.wait()
        return 0

    lax.fori_loop(0, NFULL, scatter_chunk, 0)
    tail = []
    for j in range(NFULL * DEPTH, ROWS):
        tail.append(
            pltpu.async_copy(
                vals_v.at[pl.ds(j * CHUNK, CHUNK)],
                s_sh.at[dsti_v.at[j]],
                sem,
                add=True,
            )
        )
    for d in tail:
        d.wait()

    plsc.subcore_barrier()

    pltpu.sync_copy(s_sh.at[pl.ds(0, N_NODES)], s_v)
    base = c_id * EPT2

    def edge_body(i, _):
        off = base + i * L
        vs = plsc.load_gather(s_v, [src_v[pl.ds(off, L)]])
        vd = plsc.load_gather(s_v, [dst_v[pl.ds(off, L)]])
        out_v[pl.ds(i * L, L)] = a * vs + b * vd + c
        return 0

    lax.fori_loop(0, EPT2 // L, edge_body, 0)
    pltpu.sync_copy(out_v, out_hbm.at[pl.ds(wid * EPT2, EPT2)])


def kernel(x, edge_index, W_lin, bias, W_dec, b_dec):
    ei = edge_index.astype(jnp.int32)
    src = ei[0].reshape(NS, EPT1)
    dst = ei[1].reshape(NS, EPT1)
    npad = EPT1_PAD - EPT1
    src_p = jnp.concatenate(
        [src, jnp.zeros((NS, npad), jnp.int32)], axis=1)
    dst_pad_row = N_NODES + jnp.arange(npad, dtype=jnp.int32) % (S_PAD - N_NODES)
    dst_p = jnp.concatenate(
        [dst, jnp.broadcast_to(dst_pad_row, (NS, npad))], axis=1)
    dst3 = dst_p.reshape(NS, ROWS, CHUNK)

    params = jnp.concatenate([
        W_lin.reshape(-1),
        W_dec.reshape(-1),
        bias.reshape(-1),
        jnp.pad(b_dec.reshape(-1), (0, L - 1)),
    ]).astype(jnp.float32)

    out_flat = _sc_kernel(x.reshape(-1), src_p, dst_p, dst3, params)
    return out_flat.reshape(N_EDGES, 1)


# speedup vs baseline: 46.5965x; 46.5965x over previous
import functools

import jax
import jax.numpy as jnp
from jax import lax
from jax.experimental import pallas as pl
from jax.experimental.pallas import tpu as pltpu
from jax.experimental.pallas import tpu_sc as plsc

N_NODES = 10000
N_EDGES = 320000
HIDDEN = 128

NC = 2
NS = 16
L = 16

EPT1 = N_EDGES // NS
EPT2 = N_EDGES // (NC * NS)
CHUNK = 128
ROWS = -(-EPT1 // CHUNK)
EPT1_PAD = ROWS * CHUNK
S_PAD = 10240
ZSLICE = S_PAD // NS
PARAMS = 528

_mesh = plsc.VectorSubcoreMesh(core_axis_name="c", subcore_axis_name="s")


@functools.partial(
    pl.kernel,
    out_type=jax.ShapeDtypeStruct((N_EDGES,), jnp.float32),
    mesh=_mesh,
    compiler_params=pltpu.CompilerParams(needs_layout_passes=False),
    scratch_types=[
        pltpu.VMEM((N_NODES,), jnp.float32),
        pltpu.VMEM((EPT1_PAD,), jnp.int32),
        pltpu.VMEM((EPT1_PAD,), jnp.int32),
        pltpu.VMEM((ROWS, CHUNK), jnp.int32),
        pltpu.VMEM((EPT1_PAD,), jnp.float32),
        pltpu.VMEM((N_NODES,), jnp.float32),
        pltpu.VMEM((EPT2,), jnp.float32),
        pltpu.VMEM((S_PAD // NS,), jnp.float32),
        pltpu.VMEM((PARAMS,), jnp.float32),
        pltpu.VMEM_SHARED((S_PAD,), jnp.float32),
        pltpu.SemaphoreType.DMA,
    ],
)
def _sc_kernel(x_hbm, srcp_hbm, dstp_hbm, dst3_hbm, par_hbm, out_hbm,
               x_v, src_v, dst_v, dsti_v, vals_v, s_v, out_v, zb_v, par_v,
               s_sh, sem):
    c_id = lax.axis_index("c")
    s_id = lax.axis_index("s")
    wid = 2 * s_id + c_id

    pltpu.sync_copy(x_hbm, x_v)
    pltpu.sync_copy(srcp_hbm.at[s_id], src_v)
    pltpu.sync_copy(dstp_hbm.at[s_id], dst_v)
    pltpu.sync_copy(dst3_hbm.at[s_id], dsti_v)
    pltpu.sync_copy(par_hbm, par_v)

    zeros = jnp.zeros((L,), jnp.float32)
    for k in range(ZSLICE // L):
        zb_v[pl.ds(k * L, L)] = zeros
    pltpu.sync_copy(zb_v, s_sh.at[pl.ds(s_id * ZSLICE, ZSLICE)])

    acc_a = jnp.zeros((L,), jnp.float32)
    acc_b = jnp.zeros((L,), jnp.float32)
    acc_c = jnp.zeros((L,), jnp.float32)
    for j in range(HIDDEN // L):
        w = par_v[pl.ds(j * L, L)]
        w1 = par_v[pl.ds(HIDDEN + j * L, L)]
        w2 = par_v[pl.ds(2 * HIDDEN + j * L, L)]
        bi = par_v[pl.ds(3 * HIDDEN + j * L, L)]
        acc_a = acc_a + w * w1
        acc_b = acc_b + w * w2
        acc_c = acc_c + bi * (w1 + w2)
    acc_c = acc_c + par_v[pl.ds(4 * HIDDEN, L)]

    lane = lax.iota(jnp.int32, L)
    _dnums = lax.GatherDimensionNumbers(
        offset_dims=(), collapsed_slice_dims=(0,), start_index_map=(0,))

    def lane_sum(v):
        for sh in (8, 4, 2, 1):
            perm = jnp.bitwise_xor(lane, sh)
            v = v + lax.gather(
                v, perm[:, None], dimension_numbers=_dnums, slice_sizes=(1,),
                mode=lax.GatherScatterMode.PROMISE_IN_BOUNDS)
        return v

    a = lane_sum(acc_a)
    b = lane_sum(acc_b)
    c = lane_sum(acc_c)

    def gather_body(i, _):
        idx = src_v[pl.ds(i * L, L)]
        vals_v[pl.ds(i * L, L)] = plsc.load_gather(x_v, [idx])
        return 0

    lax.fori_loop(0, EPT1_PAD // L, gather_body, 0)

    plsc.subcore_barrier()

    DEPTH = 8
    NFULL = ROWS // DEPTH

    def scatter_chunk(jc, _):
        descs = []
        for k in range(DEPTH):
            j = jc * DEPTH + k
            descs.append(
                pltpu.async_copy(
                    vals_v.at[pl.ds(j * CHUNK, CHUNK)],
                    s_sh.at[dsti_v.at[j]],
                    sem,
                    add=True,
                )
            )
        for d in descs:
            d.wait()
        return 0

    lax.fori_loop(0, NFULL, scatter_chunk, 0)
    tail = []
    for j in range(NFULL * DEPTH, ROWS):
        tail.append(
            pltpu.async_copy(
                vals_v.at[pl.ds(j * CHUNK, CHUNK)],
                s_sh.at[dsti_v.at[j]],
                sem,
                add=True,
            )
        )
    for d in tail:
        d.wait()

    plsc.subcore_barrier()

    pltpu.sync_copy(s_sh.at[pl.ds(0, N_NODES)], s_v)
    base = c_id * EPT2

    def edge_body(i, _):
        off = base + i * L
        vs = plsc.load_gather(s_v, [src_v[pl.ds(off, L)]])
        vd = plsc.load_gather(s_v, [dst_v[pl.ds(off, L)]])
        out_v[pl.ds(i * L, L)] = a * vs + b * vd + c
        return 0

    lax.fori_loop(0, EPT2 // L, edge_body, 0)
    pltpu.sync_copy(out_v, out_hbm.at[pl.ds(wid * EPT2, EPT2)])


def kernel(x, edge_index, W_lin, bias, W_dec, b_dec):
    ei = edge_index.astype(jnp.int32)
    src = ei[0].reshape(NS, EPT1)
    dst = ei[1].reshape(NS, EPT1)
    npad = EPT1_PAD - EPT1
    src_p = jnp.concatenate(
        [src, jnp.zeros((NS, npad), jnp.int32)], axis=1)
    dst_pad_row = N_NODES + jnp.arange(npad, dtype=jnp.int32) % (S_PAD - N_NODES)
    dst_p = jnp.concatenate(
        [dst, jnp.broadcast_to(dst_pad_row, (NS, npad))], axis=1)
    dst3 = dst_p.reshape(NS, ROWS, CHUNK)

    params = jnp.concatenate([
        W_lin.reshape(-1),
        W_dec.reshape(-1),
        bias.reshape(-1),
        jnp.pad(b_dec.reshape(-1), (0, L - 1)),
    ]).astype(jnp.float32)

    out_flat = _sc_kernel(x.reshape(-1), src_p, dst_p, dst3, params)
    return out_flat.reshape(N_EDGES, 1)


# speedup vs baseline: 46.7999x; 1.0044x over previous
import functools

import jax
import jax.numpy as jnp
from jax import lax
from jax.experimental import pallas as pl
from jax.experimental.pallas import tpu as pltpu
from jax.experimental.pallas import tpu_sc as plsc

N_NODES = 10000
N_EDGES = 320000
HIDDEN = 128

NC = 1
NS = 16
L = 16

EPT1 = N_EDGES // NS
EPT2 = N_EDGES // (NC * NS)
CHUNK = 128
ROWS = -(-EPT1 // CHUNK)
EPT1_PAD = ROWS * CHUNK
S_PAD = 10240
ZSLICE = S_PAD // NS
PARAMS = 528

_mesh = plsc.VectorSubcoreMesh(
    core_axis_name="c", subcore_axis_name="s", num_cores=NC)


@functools.partial(
    pl.kernel,
    out_type=jax.ShapeDtypeStruct((N_EDGES,), jnp.float32),
    mesh=_mesh,
    compiler_params=pltpu.CompilerParams(needs_layout_passes=False),
    scratch_types=[
        pltpu.VMEM((N_NODES,), jnp.float32),
        pltpu.VMEM((EPT1_PAD,), jnp.int32),
        pltpu.VMEM((EPT1_PAD,), jnp.int32),
        pltpu.VMEM((ROWS, CHUNK), jnp.int32),
        pltpu.VMEM((EPT1_PAD,), jnp.float32),
        pltpu.VMEM((N_NODES,), jnp.float32),
        pltpu.VMEM((EPT2,), jnp.float32),
        pltpu.VMEM((S_PAD // NS,), jnp.float32),
        pltpu.VMEM((PARAMS,), jnp.float32),
        pltpu.VMEM_SHARED((S_PAD,), jnp.float32),
        pltpu.SemaphoreType.DMA,
    ],
)
def _sc_kernel(x_hbm, srcp_hbm, dstp_hbm, dst3_hbm, par_hbm, out_hbm,
               x_v, src_v, dst_v, dsti_v, vals_v, s_v, out_v, zb_v, par_v,
               s_sh, sem):
    c_id = lax.axis_index("c")
    s_id = lax.axis_index("s")
    wid = NC * s_id + c_id

    pltpu.sync_copy(x_hbm, x_v)
    pltpu.sync_copy(srcp_hbm.at[s_id], src_v)
    pltpu.sync_copy(dstp_hbm.at[s_id], dst_v)
    pltpu.sync_copy(dst3_hbm.at[s_id], dsti_v)
    pltpu.sync_copy(par_hbm, par_v)

    zeros = jnp.zeros((L,), jnp.float32)
    for k in range(ZSLICE // L):
        zb_v[pl.ds(k * L, L)] = zeros
    pltpu.sync_copy(zb_v, s_sh.at[pl.ds(s_id * ZSLICE, ZSLICE)])

    acc_a = jnp.zeros((L,), jnp.float32)
    acc_b = jnp.zeros((L,), jnp.float32)
    acc_c = jnp.zeros((L,), jnp.float32)
    for j in range(HIDDEN // L):
        w = par_v[pl.ds(j * L, L)]
        w1 = par_v[pl.ds(HIDDEN + j * L, L)]
        w2 = par_v[pl.ds(2 * HIDDEN + j * L, L)]
        bi = par_v[pl.ds(3 * HIDDEN + j * L, L)]
        acc_a = acc_a + w * w1
        acc_b = acc_b + w * w2
        acc_c = acc_c + bi * (w1 + w2)
    acc_c = acc_c + par_v[pl.ds(4 * HIDDEN, L)]

    lane = lax.iota(jnp.int32, L)
    _dnums = lax.GatherDimensionNumbers(
        offset_dims=(), collapsed_slice_dims=(0,), start_index_map=(0,))

    def lane_sum(v):
        for sh in (8, 4, 2, 1):
            perm = jnp.bitwise_xor(lane, sh)
            v = v + lax.gather(
                v, perm[:, None], dimension_numbers=_dnums, slice_sizes=(1,),
                mode=lax.GatherScatterMode.PROMISE_IN_BOUNDS)
        return v

    a = lane_sum(acc_a)
    b = lane_sum(acc_b)
    c = lane_sum(acc_c)

    def gather_body(i, _):
        idx = src_v[pl.ds(i * L, L)]
        vals_v[pl.ds(i * L, L)] = plsc.load_gather(x_v, [idx])
        return 0

    lax.fori_loop(0, EPT1_PAD // L, gather_body, 0)

    plsc.subcore_barrier()

    DEPTH = 8
    NFULL = ROWS // DEPTH

    def scatter_chunk(jc, _):
        descs = []
        for k in range(DEPTH):
            j = jc * DEPTH + k
            descs.append(
                pltpu.async_copy(
                    vals_v.at[pl.ds(j * CHUNK, CHUNK)],
                    s_sh.at[dsti_v.at[j]],
                    sem,
                    add=True,
                )
            )
        for d in descs:
            d.wait()
        return 0

    lax.fori_loop(0, NFULL, scatter_chunk, 0)
    tail = []
    for j in range(NFULL * DEPTH, ROWS):
        tail.append(
            pltpu.async_copy(
                vals_v.at[pl.ds(j * CHUNK, CHUNK)],
                s_sh.at[dsti_v.at[j]],
                sem,
                add=True,
            )
        )
    for d in tail:
        d.wait()

    plsc.subcore_barrier()

    pltpu.sync_copy(s_sh.at[pl.ds(0, N_NODES)], s_v)
    base = c_id * EPT2

    def edge_body(i, _):
        off = base + i * L
        vs = plsc.load_gather(s_v, [src_v[pl.ds(off, L)]])
        vd = plsc.load_gather(s_v, [dst_v[pl.ds(off, L)]])
        out_v[pl.ds(i * L, L)] = a * vs + b * vd + c
        return 0

    lax.fori_loop(0, EPT2 // L, edge_body, 0)
    pltpu.sync_copy(out_v, out_hbm.at[pl.ds(wid * EPT2, EPT2)])


def kernel(x, edge_index, W_lin, bias, W_dec, b_dec):
    ei = edge_index.astype(jnp.int32)
    src = ei[0].reshape(NS, EPT1)
    dst = ei[1].reshape(NS, EPT1)
    npad = EPT1_PAD - EPT1
    src_p = jnp.concatenate(
        [src, jnp.zeros((NS, npad), jnp.int32)], axis=1)
    dst_pad_row = N_NODES + jnp.arange(npad, dtype=jnp.int32) % (S_PAD - N_NODES)
    dst_p = jnp.concatenate(
        [dst, jnp.broadcast_to(dst_pad_row, (NS, npad))], axis=1)
    dst3 = dst_p.reshape(NS, ROWS, CHUNK)

    params = jnp.concatenate([
        W_lin.reshape(-1),
        W_dec.reshape(-1),
        bias.reshape(-1),
        jnp.pad(b_dec.reshape(-1), (0, L - 1)),
    ]).astype(jnp.float32)

    out_flat = _sc_kernel(x.reshape(-1), src_p, dst_p, dst3, params)
    return out_flat.reshape(N_EDGES, 1)
